# baseline (speedup 1.0000x reference)
import functools

import jax
import jax.numpy as jnp
import numpy as np
from jax import lax
from jax.experimental import pallas as pl
from jax.experimental.pallas import tpu as pltpu
from jax.experimental.pallas import tpu_sc as plsc

N_NODES = 10000
N_EDGES = 320000
D_NODE = 128
D_EDGE = 16

_NC, _NS = 2, 16
_NW = _NC * _NS
_CH = 128
_NCHUNK = N_EDGES // _CH
_SLOTS = -(-_NCHUNK // _NW)
_ZCH = 200
_NZ = N_NODES // _ZCH
_ZSLOTS = -(-_NZ // _NS)

_MESH = dict(core_axis_name="c", subcore_axis_name="s")


def _sc_gather(node_features, row, col):
    @functools.partial(
        pl.kernel,
        out_type=(
            jax.ShapeDtypeStruct((N_EDGES, D_NODE), jnp.float32),
            jax.ShapeDtypeStruct((N_EDGES, D_NODE), jnp.float32),
        ),
        scratch_types=[
            pltpu.VMEM((_CH,), jnp.int32),
            pltpu.VMEM((_CH,), jnp.int32),
            pltpu.VMEM((_CH, D_NODE), jnp.float32),
            pltpu.VMEM((_CH, D_NODE), jnp.float32),
            pltpu.SemaphoreType.DMA,
        ],
        mesh=plsc.VectorSubcoreMesh(**_MESH),
    )
    def gather_k(nf, row_h, col_h, src_o, tgt_o, idx_r, idx_c, buf_s, buf_t, sem):
        wid = lax.axis_index("s") * _NC + lax.axis_index("c")

        def body(g, c):
            chunk = wid + _NW * g

            @pl.when(chunk < _NCHUNK)
            def _():
                base = chunk * _CH
                pltpu.sync_copy(row_h.at[pl.ds(base, _CH)], idx_r)
                pltpu.sync_copy(col_h.at[pl.ds(base, _CH)], idx_c)
                a = pltpu.async_copy(nf.at[idx_r], buf_s, sem)
                b = pltpu.async_copy(nf.at[idx_c], buf_t, sem)
                a.wait()
                b.wait()
                pltpu.sync_copy(buf_s, src_o.at[pl.ds(base, _CH)])
                pltpu.sync_copy(buf_t, tgt_o.at[pl.ds(base, _CH)])

            return c

        lax.fori_loop(0, _SLOTS, body, 0)

    return gather_k(node_features, row, col)


def _sc_scatter(e_out, col):
    @functools.partial(
        pl.kernel,
        out_type=jax.ShapeDtypeStruct((_NC, N_NODES, D_NODE), jnp.float32),
        scratch_types=[
            pltpu.VMEM((_CH,), jnp.int32),
            pltpu.VMEM((_CH, D_NODE), jnp.float32),
            pltpu.VMEM((_ZCH, D_NODE), jnp.float32),
            pltpu.VMEM_SHARED((N_NODES, D_NODE), jnp.float32),
        ],
        mesh=plsc.VectorSubcoreMesh(**_MESH),
    )
    def scatter_k(eout_h, col_h, agg_o, idx_c, buf, zbuf, agg_sh):
        cid = lax.axis_index("c")
        sid = lax.axis_index("s")
        wid = sid * _NC + cid

        def zrow(i, c):
            def zcol(j, c2):
                zbuf[i, pl.ds(j * 16, 16)] = jnp.zeros((16,), jnp.float32)
                return c2

            return lax.fori_loop(0, D_NODE // 16, zcol, c)

        lax.fori_loop(0, _ZCH, zrow, 0)

        def zcp(k, c):
            zc = sid + _NS * k

            @pl.when(zc < _NZ)
            def _():
                pltpu.sync_copy(zbuf, agg_sh.at[pl.ds(zc * _ZCH, _ZCH)])

            return c

        lax.fori_loop(0, _ZSLOTS, zcp, 0)
        plsc.subcore_barrier()

        def body(g, c):
            chunk = wid + _NW * g

            @pl.when(chunk < _NCHUNK)
            def _():
                base = chunk * _CH
                pltpu.sync_copy(col_h.at[pl.ds(base, _CH)], idx_c)
                pltpu.sync_copy(eout_h.at[pl.ds(base, _CH)], buf)
                pltpu.sync_copy(buf, agg_sh.at[idx_c], add=True)

            return c

        lax.fori_loop(0, _SLOTS, body, 0)
        plsc.subcore_barrier()

        def cp(k, c):
            zc = sid + _NS * k

            @pl.when(zc < _NZ)
            def _():
                off = zc * _ZCH
                pltpu.sync_copy(agg_sh.at[pl.ds(off, _ZCH)], zbuf)
                pltpu.sync_copy(zbuf, agg_o.at[cid, pl.ds(off, _ZCH)])

            return c

        lax.fori_loop(0, _ZSLOTS, cp, 0)

    return scatter_k(e_out, col)


def _rowsum(x):
    ones = jnp.ones((x.shape[1], 8), jnp.float32)
    return jnp.dot(x, ones, preferred_element_type=jnp.float32)[:, :1]


def _ln_gelu(x, g, b):
    m = _rowsum(x) * (1.0 / x.shape[1])
    c = x - m
    v = _rowsum(c * c) * (1.0 / x.shape[1])
    y = c * lax.rsqrt(v + 1e-5) * g + b
    return 0.5 * y * (1.0 + lax.erf(y * (1.0 / np.sqrt(2.0))))


_BE = 512


def _edge_body(src, tgt, ef, wsd, wtd, we, wdc, eb1, eg1, ebt1,
               ew2, eb2, eg2, ebt2, out):
    s = src[...]
    t = tgt[...]
    st = _rowsum(s * t)
    ss = _rowsum(s * s)
    tt = _rowsum(t * t)
    d2 = jnp.maximum(ss + tt - 2.0 * st, 0.0)
    dist = jnp.sqrt(d2 + 1e-12)
    cos = st / (jnp.sqrt(ss + 1e-12) * jnp.sqrt(tt + 1e-12))
    h = jnp.dot(s, wsd[...], preferred_element_type=jnp.float32)
    h = h + jnp.dot(t, wtd[...], preferred_element_type=jnp.float32)
    h = h + jnp.dot(ef[...], we[...], preferred_element_type=jnp.float32)
    h = h + dist * wdc[0:1, :] + cos * wdc[1:2, :] + eb1[...]
    h = _ln_gelu(h, eg1[...], ebt1[...])
    h2 = jnp.dot(h, ew2[...], preferred_element_type=jnp.float32) + eb2[...]
    out[...] = _ln_gelu(h2, eg2[...], ebt2[...])


def _edge_mlp(src, tgt, ef, wsd, wtd, we, wdc, eb1, eg1, ebt1,
              ew2, eb2, eg2, ebt2):
    n_blk = N_EDGES // _BE
    row_spec = pl.BlockSpec((_BE, D_NODE), lambda i: (i, 0))
    const = lambda shape: pl.BlockSpec(shape, lambda i: (0, 0))
    return pl.pallas_call(
        _edge_body,
        grid=(n_blk,),
        in_specs=[
            row_spec,
            row_spec,
            pl.BlockSpec((_BE, D_EDGE), lambda i: (i, 0)),
            const((D_NODE, D_NODE)),
            const((D_NODE, D_NODE)),
            const((D_EDGE, D_NODE)),
            const((2, D_NODE)),
            const((1, D_NODE)),
            const((1, D_NODE)),
            const((1, D_NODE)),
            const((D_NODE, D_NODE)),
            const((1, D_NODE)),
            const((1, D_NODE)),
            const((1, D_NODE)),
        ],
        out_specs=row_spec,
        out_shape=jax.ShapeDtypeStruct((N_EDGES, D_NODE), jnp.float32),
    )(src, tgt, ef, wsd, wtd, we, wdc, eb1, eg1, ebt1, ew2, eb2, eg2, ebt2)


_BN = 1000


def _node_body(nf, agg, nwx, nwa, nb1, ng1, nbt1, nw2, nb2, ng2, nbt2, out):
    x = nf[...]
    a = agg[0] + agg[1]
    h = jnp.dot(x, nwx[...], preferred_element_type=jnp.float32)
    h = h + jnp.dot(a, nwa[...], preferred_element_type=jnp.float32) + nb1[...]
    h = _ln_gelu(h, ng1[...], nbt1[...])
    h2 = jnp.dot(h, nw2[...], preferred_element_type=jnp.float32) + nb2[...]
    out[...] = _ln_gelu(h2, ng2[...], nbt2[...])


def _node_mlp(nf, agg2, nwx, nwa, nb1, ng1, nbt1, nw2, nb2, ng2, nbt2):
    n_blk = N_NODES // _BN
    row_spec = pl.BlockSpec((_BN, D_NODE), lambda i: (i, 0))
    const = lambda shape: pl.BlockSpec(shape, lambda i: (0, 0))
    return pl.pallas_call(
        _node_body,
        grid=(n_blk,),
        in_specs=[
            row_spec,
            pl.BlockSpec((_NC, _BN, D_NODE), lambda i: (0, i, 0)),
            const((D_NODE, D_NODE)),
            const((D_NODE, D_NODE)),
            const((1, D_NODE)),
            const((1, D_NODE)),
            const((1, D_NODE)),
            const((D_NODE, D_NODE)),
            const((1, D_NODE)),
            const((1, D_NODE)),
            const((1, D_NODE)),
        ],
        out_specs=row_spec,
        out_shape=jax.ShapeDtypeStruct((N_NODES, D_NODE), jnp.float32),
    )(nf, agg2, nwx, nwa, nb1, ng1, nbt1, nw2, nb2, ng2, nbt2)


def kernel(node_features, edge_features,
           eW1, eb1, eg1, ebt1, eW2, eb2, eg2, ebt2,
           nW1, nb1, ng1, nbt1, nW2, nb2, ng2, nbt2,
           edge_index):
    ei = edge_index.astype(jnp.int32)
    row = ei[0]
    col = ei[1]

    wsd = eW1[0:D_NODE] + eW1[2 * D_NODE:3 * D_NODE]
    wtd = eW1[D_NODE:2 * D_NODE] - eW1[2 * D_NODE:3 * D_NODE]
    wdc = eW1[3 * D_NODE:3 * D_NODE + 2]
    we = eW1[3 * D_NODE + 2:]

    src, tgt = _sc_gather(node_features, row, col)
    e_out = _edge_mlp(
        src, tgt, edge_features, wsd, wtd, we, wdc,
        eb1.reshape(1, -1), eg1.reshape(1, -1), ebt1.reshape(1, -1),
        eW2, eb2.reshape(1, -1), eg2.reshape(1, -1), ebt2.reshape(1, -1))
    agg2 = _sc_scatter(e_out, col)
    x_out = _node_mlp(
        node_features, agg2, nW1[0:D_NODE], nW1[D_NODE:],
        nb1.reshape(1, -1), ng1.reshape(1, -1), nbt1.reshape(1, -1),
        nW2, nb2.reshape(1, -1), ng2.reshape(1, -1), nbt2.reshape(1, -1))
    return (x_out, e_out)


# speedup vs baseline: 1.4250x; 1.4250x over previous
import functools

import jax
import jax.numpy as jnp
import numpy as np
from jax import lax
from jax.experimental import pallas as pl
from jax.experimental.pallas import tpu as pltpu
from jax.experimental.pallas import tpu_sc as plsc

N_NODES = 10000
N_EDGES = 320000
D_NODE = 128
D_EDGE = 16

_NC, _NS = 2, 16
_NW = _NC * _NS
_CH = 128
_NCHUNK = N_EDGES // _CH
_SLOTS = -(-_NCHUNK // _NW)
_ZCH = 200
_NZ = N_NODES // _ZCH
_ZSLOTS = -(-_NZ // _NS)

_MESH = dict(core_axis_name="c", subcore_axis_name="s")


def _sc_gather(node_features, row, col):
    @functools.partial(
        pl.kernel,
        out_type=(
            jax.ShapeDtypeStruct((N_EDGES, D_NODE), jnp.float32),
            jax.ShapeDtypeStruct((N_EDGES, D_NODE), jnp.float32),
        ),
        scratch_types=[
            pltpu.VMEM((_CH,), jnp.int32),
            pltpu.VMEM((_CH,), jnp.int32),
            pltpu.VMEM((_CH, D_NODE), jnp.float32),
            pltpu.VMEM((_CH, D_NODE), jnp.float32),
            pltpu.SemaphoreType.DMA,
        ],
        mesh=plsc.VectorSubcoreMesh(**_MESH),
    )
    def gather_k(nf, row_h, col_h, src_o, tgt_o, idx_r, idx_c, buf_s, buf_t, sem):
        wid = lax.axis_index("s") * _NC + lax.axis_index("c")

        def body(g, c):
            chunk = wid + _NW * g

            @pl.when(chunk < _NCHUNK)
            def _():
                base = chunk * _CH
                pltpu.sync_copy(row_h.at[pl.ds(base, _CH)], idx_r)
                pltpu.sync_copy(col_h.at[pl.ds(base, _CH)], idx_c)
                a = pltpu.async_copy(nf.at[idx_r], buf_s, sem)
                b = pltpu.async_copy(nf.at[idx_c], buf_t, sem)
                a.wait()
                b.wait()
                pltpu.sync_copy(buf_s, src_o.at[pl.ds(base, _CH)])
                pltpu.sync_copy(buf_t, tgt_o.at[pl.ds(base, _CH)])

            return c

        lax.fori_loop(0, _SLOTS, body, 0)

    return gather_k(node_features, row, col)


def _sc_scatter(e_out, col):
    @functools.partial(
        pl.kernel,
        out_type=jax.ShapeDtypeStruct((_NC, N_NODES, D_NODE), jnp.float32),
        scratch_types=[
            pltpu.VMEM((_CH,), jnp.int32),
            pltpu.VMEM((_CH, D_NODE), jnp.float32),
            pltpu.VMEM((_ZCH, D_NODE), jnp.float32),
            pltpu.VMEM_SHARED((N_NODES, D_NODE), jnp.float32),
        ],
        mesh=plsc.VectorSubcoreMesh(**_MESH),
    )
    def scatter_k(eout_h, col_h, agg_o, idx_c, buf, zbuf, agg_sh):
        cid = lax.axis_index("c")
        sid = lax.axis_index("s")
        wid = sid * _NC + cid

        def zrow(i, c):
            def zcol(j, c2):
                zbuf[i, pl.ds(j * 16, 16)] = jnp.zeros((16,), jnp.float32)
                return c2

            return lax.fori_loop(0, D_NODE // 16, zcol, c)

        lax.fori_loop(0, _ZCH, zrow, 0)

        def zcp(k, c):
            zc = sid + _NS * k

            @pl.when(zc < _NZ)
            def _():
                pltpu.sync_copy(zbuf, agg_sh.at[pl.ds(zc * _ZCH, _ZCH)])

            return c

        lax.fori_loop(0, _ZSLOTS, zcp, 0)
        plsc.subcore_barrier()

        def body(g, c):
            chunk = wid + _NW * g

            @pl.when(chunk < _NCHUNK)
            def _():
                base = chunk * _CH
                pltpu.sync_copy(col_h.at[pl.ds(base, _CH)], idx_c)
                pltpu.sync_copy(eout_h.at[pl.ds(base, _CH)], buf)
                pltpu.sync_copy(buf, agg_sh.at[idx_c], add=True)

            return c

        lax.fori_loop(0, _SLOTS, body, 0)
        plsc.subcore_barrier()

        def cp(k, c):
            zc = sid + _NS * k

            @pl.when(zc < _NZ)
            def _():
                off = zc * _ZCH
                pltpu.sync_copy(agg_sh.at[pl.ds(off, _ZCH)], zbuf)
                pltpu.sync_copy(zbuf, agg_o.at[cid, pl.ds(off, _ZCH)])

            return c

        lax.fori_loop(0, _ZSLOTS, cp, 0)

    return scatter_k(e_out, col)


def _rowsum(x):
    return jnp.sum(x, axis=1, keepdims=True)


def _ln_gelu(x, g, b):
    m = _rowsum(x) * (1.0 / x.shape[1])
    c = x - m
    v = _rowsum(c * c) * (1.0 / x.shape[1])
    y = c * lax.rsqrt(v + 1e-5) * g + b
    return 0.5 * y * (1.0 + lax.erf(y * (1.0 / np.sqrt(2.0))))


_BE = 2000


def _edge_body(src, tgt, ef, wsd, wtd, we, wdc, eb1, eg1, ebt1,
               ew2, eb2, eg2, ebt2, out):
    s = src[...]
    t = tgt[...]
    st = _rowsum(s * t)
    ss = _rowsum(s * s)
    tt = _rowsum(t * t)
    d2 = jnp.maximum(ss + tt - 2.0 * st, 0.0)
    dist = jnp.sqrt(d2 + 1e-12)
    cos = st / (jnp.sqrt(ss + 1e-12) * jnp.sqrt(tt + 1e-12))
    h = jnp.dot(s, wsd[...], preferred_element_type=jnp.float32)
    h = h + jnp.dot(t, wtd[...], preferred_element_type=jnp.float32)
    h = h + jnp.dot(ef[...], we[...], preferred_element_type=jnp.float32)
    h = h + dist * wdc[0:1, :] + cos * wdc[1:2, :] + eb1[...]
    h = _ln_gelu(h, eg1[...], ebt1[...])
    h2 = jnp.dot(h, ew2[...], preferred_element_type=jnp.float32) + eb2[...]
    out[...] = _ln_gelu(h2, eg2[...], ebt2[...])


def _edge_mlp(src, tgt, ef, wsd, wtd, we, wdc, eb1, eg1, ebt1,
              ew2, eb2, eg2, ebt2):
    n_blk = N_EDGES // _BE
    row_spec = pl.BlockSpec((_BE, D_NODE), lambda i: (i, 0))
    const = lambda shape: pl.BlockSpec(shape, lambda i: (0, 0))
    return pl.pallas_call(
        _edge_body,
        grid=(n_blk,),
        in_specs=[
            row_spec,
            row_spec,
            pl.BlockSpec((_BE, D_EDGE), lambda i: (i, 0)),
            const((D_NODE, D_NODE)),
            const((D_NODE, D_NODE)),
            const((D_EDGE, D_NODE)),
            const((2, D_NODE)),
            const((1, D_NODE)),
            const((1, D_NODE)),
            const((1, D_NODE)),
            const((D_NODE, D_NODE)),
            const((1, D_NODE)),
            const((1, D_NODE)),
            const((1, D_NODE)),
        ],
        out_specs=row_spec,
        out_shape=jax.ShapeDtypeStruct((N_EDGES, D_NODE), jnp.float32),
    )(src, tgt, ef, wsd, wtd, we, wdc, eb1, eg1, ebt1, ew2, eb2, eg2, ebt2)


_BN = 1000


def _node_body(nf, agg, nwx, nwa, nb1, ng1, nbt1, nw2, nb2, ng2, nbt2, out):
    x = nf[...]
    a = agg[0] + agg[1]
    h = jnp.dot(x, nwx[...], preferred_element_type=jnp.float32)
    h = h + jnp.dot(a, nwa[...], preferred_element_type=jnp.float32) + nb1[...]
    h = _ln_gelu(h, ng1[...], nbt1[...])
    h2 = jnp.dot(h, nw2[...], preferred_element_type=jnp.float32) + nb2[...]
    out[...] = _ln_gelu(h2, ng2[...], nbt2[...])


def _node_mlp(nf, agg2, nwx, nwa, nb1, ng1, nbt1, nw2, nb2, ng2, nbt2):
    n_blk = N_NODES // _BN
    row_spec = pl.BlockSpec((_BN, D_NODE), lambda i: (i, 0))
    const = lambda shape: pl.BlockSpec(shape, lambda i: (0, 0))
    return pl.pallas_call(
        _node_body,
        grid=(n_blk,),
        in_specs=[
            row_spec,
            pl.BlockSpec((_NC, _BN, D_NODE), lambda i: (0, i, 0)),
            const((D_NODE, D_NODE)),
            const((D_NODE, D_NODE)),
            const((1, D_NODE)),
            const((1, D_NODE)),
            const((1, D_NODE)),
            const((D_NODE, D_NODE)),
            const((1, D_NODE)),
            const((1, D_NODE)),
            const((1, D_NODE)),
        ],
        out_specs=row_spec,
        out_shape=jax.ShapeDtypeStruct((N_NODES, D_NODE), jnp.float32),
    )(nf, agg2, nwx, nwa, nb1, ng1, nbt1, nw2, nb2, ng2, nbt2)


def kernel(node_features, edge_features,
           eW1, eb1, eg1, ebt1, eW2, eb2, eg2, ebt2,
           nW1, nb1, ng1, nbt1, nW2, nb2, ng2, nbt2,
           edge_index):
    ei = edge_index.astype(jnp.int32)
    row = ei[0]
    col = ei[1]

    wsd = eW1[0:D_NODE] + eW1[2 * D_NODE:3 * D_NODE]
    wtd = eW1[D_NODE:2 * D_NODE] - eW1[2 * D_NODE:3 * D_NODE]
    wdc = eW1[3 * D_NODE:3 * D_NODE + 2]
    we = eW1[3 * D_NODE + 2:]

    src, tgt = _sc_gather(node_features, row, col)
    e_out = _edge_mlp(
        src, tgt, edge_features, wsd, wtd, we, wdc,
        eb1.reshape(1, -1), eg1.reshape(1, -1), ebt1.reshape(1, -1),
        eW2, eb2.reshape(1, -1), eg2.reshape(1, -1), ebt2.reshape(1, -1))
    agg2 = _sc_scatter(e_out, col)
    x_out = _node_mlp(
        node_features, agg2, nW1[0:D_NODE], nW1[D_NODE:],
        nb1.reshape(1, -1), ng1.reshape(1, -1), nbt1.reshape(1, -1),
        nW2, nb2.reshape(1, -1), ng2.reshape(1, -1), nbt2.reshape(1, -1))
    return (x_out, e_out)


# speedup vs baseline: 1.6515x; 1.1589x over previous
import functools

import jax
import jax.numpy as jnp
import numpy as np
from jax import lax
from jax.experimental import pallas as pl
from jax.experimental.pallas import tpu as pltpu
from jax.experimental.pallas import tpu_sc as plsc

N_NODES = 10000
N_EDGES = 320000
D_NODE = 128
D_EDGE = 16

_NC, _NS = 2, 16
_NW = _NC * _NS
_CH = 128
_NCHUNK = N_EDGES // _CH
_SLOTS = -(-_NCHUNK // _NW)
_ZCH = 200
_NZ = N_NODES // _ZCH
_ZSLOTS = -(-_NZ // _NS)

_MESH = dict(core_axis_name="c", subcore_axis_name="s")


def _sc_gather(node_features, row, col):
    n_e = row.shape[0]
    n_chunk = n_e // _CH
    slots = -(-n_chunk // _NW)

    @functools.partial(
        pl.kernel,
        out_type=(
            jax.ShapeDtypeStruct((n_e, D_NODE), jnp.float32),
            jax.ShapeDtypeStruct((n_e, D_NODE), jnp.float32),
        ),
        scratch_types=[
            pltpu.VMEM((_CH,), jnp.int32),
            pltpu.VMEM((_CH,), jnp.int32),
            pltpu.VMEM((_CH, D_NODE), jnp.float32),
            pltpu.VMEM((_CH, D_NODE), jnp.float32),
            pltpu.SemaphoreType.DMA,
        ],
        mesh=plsc.VectorSubcoreMesh(**_MESH),
    )
    def gather_k(nf, row_h, col_h, src_o, tgt_o, idx_r, idx_c, buf_s, buf_t, sem):
        wid = lax.axis_index("s") * _NC + lax.axis_index("c")

        def body(g, c):
            chunk = wid + _NW * g

            @pl.when(chunk < n_chunk)
            def _():
                base = chunk * _CH
                pltpu.sync_copy(row_h.at[pl.ds(base, _CH)], idx_r)
                pltpu.sync_copy(col_h.at[pl.ds(base, _CH)], idx_c)
                a = pltpu.async_copy(nf.at[idx_r], buf_s, sem)
                b = pltpu.async_copy(nf.at[idx_c], buf_t, sem)
                a.wait()
                b.wait()
                pltpu.sync_copy(buf_s, src_o.at[pl.ds(base, _CH)])
                pltpu.sync_copy(buf_t, tgt_o.at[pl.ds(base, _CH)])

            return c

        lax.fori_loop(0, slots, body, 0)

    return gather_k(node_features, row, col)


def _sc_scatter(e_out, col):
    n_e = col.shape[0]
    n_chunk = n_e // _CH
    slots = -(-n_chunk // _NW)

    @functools.partial(
        pl.kernel,
        out_type=jax.ShapeDtypeStruct((_NC, N_NODES, D_NODE), jnp.float32),
        scratch_types=[
            pltpu.VMEM((_CH,), jnp.int32),
            pltpu.VMEM((_CH, D_NODE), jnp.float32),
            pltpu.VMEM((_ZCH, D_NODE), jnp.float32),
            pltpu.VMEM_SHARED((N_NODES, D_NODE), jnp.float32),
        ],
        mesh=plsc.VectorSubcoreMesh(**_MESH),
    )
    def scatter_k(eout_h, col_h, agg_o, idx_c, buf, zbuf, agg_sh):
        cid = lax.axis_index("c")
        sid = lax.axis_index("s")
        wid = sid * _NC + cid

        def zrow(i, c):
            def zcol(j, c2):
                zbuf[i, pl.ds(j * 16, 16)] = jnp.zeros((16,), jnp.float32)
                return c2

            return lax.fori_loop(0, D_NODE // 16, zcol, c)

        lax.fori_loop(0, _ZCH, zrow, 0)

        def zcp(k, c):
            zc = sid + _NS * k

            @pl.when(zc < _NZ)
            def _():
                pltpu.sync_copy(zbuf, agg_sh.at[pl.ds(zc * _ZCH, _ZCH)])

            return c

        lax.fori_loop(0, _ZSLOTS, zcp, 0)
        plsc.subcore_barrier()

        def body(g, c):
            chunk = wid + _NW * g

            @pl.when(chunk < n_chunk)
            def _():
                base = chunk * _CH
                pltpu.sync_copy(col_h.at[pl.ds(base, _CH)], idx_c)
                pltpu.sync_copy(eout_h.at[pl.ds(base, _CH)], buf)
                pltpu.sync_copy(buf, agg_sh.at[idx_c], add=True)

            return c

        lax.fori_loop(0, slots, body, 0)
        plsc.subcore_barrier()

        def cp(k, c):
            zc = sid + _NS * k

            @pl.when(zc < _NZ)
            def _():
                off = zc * _ZCH
                pltpu.sync_copy(agg_sh.at[pl.ds(off, _ZCH)], zbuf)
                pltpu.sync_copy(zbuf, agg_o.at[cid, pl.ds(off, _ZCH)])

            return c

        lax.fori_loop(0, _ZSLOTS, cp, 0)

    return scatter_k(e_out, col)


def _rowsum(x):
    return jnp.sum(x, axis=1, keepdims=True)


def _ln_gelu(x, g, b):
    m = _rowsum(x) * (1.0 / x.shape[1])
    c = x - m
    v = _rowsum(c * c) * (1.0 / x.shape[1])
    y = c * lax.rsqrt(v + 1e-5) * g + b
    return 0.5 * y * (1.0 + lax.erf(y * (1.0 / np.sqrt(2.0))))


_BE = 2000


def _edge_body(src, tgt, ef, wsd, wtd, we, wdc, eb1, eg1, ebt1,
               ew2, eb2, eg2, ebt2, out):
    s = src[...]
    t = tgt[...]
    st = _rowsum(s * t)
    ss = _rowsum(s * s)
    tt = _rowsum(t * t)
    d2 = jnp.maximum(ss + tt - 2.0 * st, 0.0)
    dist = jnp.sqrt(d2 + 1e-12)
    cos = st / (jnp.sqrt(ss + 1e-12) * jnp.sqrt(tt + 1e-12))
    h = jnp.dot(s, wsd[...], preferred_element_type=jnp.float32)
    h = h + jnp.dot(t, wtd[...], preferred_element_type=jnp.float32)
    h = h + jnp.dot(ef[...], we[...], preferred_element_type=jnp.float32)
    h = h + dist * wdc[0:1, :] + cos * wdc[1:2, :] + eb1[...]
    h = _ln_gelu(h, eg1[...], ebt1[...])
    h2 = jnp.dot(h, ew2[...], preferred_element_type=jnp.float32) + eb2[...]
    out[...] = _ln_gelu(h2, eg2[...], ebt2[...])


def _edge_mlp(src, tgt, ef, wsd, wtd, we, wdc, eb1, eg1, ebt1,
              ew2, eb2, eg2, ebt2):
    n_blk = src.shape[0] // _BE
    row_spec = pl.BlockSpec((_BE, D_NODE), lambda i: (i, 0))
    const = lambda shape: pl.BlockSpec(shape, lambda i: (0, 0))
    return pl.pallas_call(
        _edge_body,
        grid=(n_blk,),
        in_specs=[
            row_spec,
            row_spec,
            pl.BlockSpec((_BE, D_EDGE), lambda i: (i, 0)),
            const((D_NODE, D_NODE)),
            const((D_NODE, D_NODE)),
            const((D_EDGE, D_NODE)),
            const((2, D_NODE)),
            const((1, D_NODE)),
            const((1, D_NODE)),
            const((1, D_NODE)),
            const((D_NODE, D_NODE)),
            const((1, D_NODE)),
            const((1, D_NODE)),
            const((1, D_NODE)),
        ],
        out_specs=row_spec,
        out_shape=jax.ShapeDtypeStruct((src.shape[0], D_NODE), jnp.float32),
    )(src, tgt, ef, wsd, wtd, we, wdc, eb1, eg1, ebt1, ew2, eb2, eg2, ebt2)


_BN = 1000


def _node_body(nf, agg_a, agg_b, nwx, nwa, nb1, ng1, nbt1, nw2, nb2, ng2, nbt2, out):
    x = nf[...]
    a = agg_a[0] + agg_a[1] + agg_b[0] + agg_b[1]
    h = jnp.dot(x, nwx[...], preferred_element_type=jnp.float32)
    h = h + jnp.dot(a, nwa[...], preferred_element_type=jnp.float32) + nb1[...]
    h = _ln_gelu(h, ng1[...], nbt1[...])
    h2 = jnp.dot(h, nw2[...], preferred_element_type=jnp.float32) + nb2[...]
    out[...] = _ln_gelu(h2, ng2[...], nbt2[...])


def _node_mlp(nf, agg_a, agg_b, nwx, nwa, nb1, ng1, nbt1, nw2, nb2, ng2, nbt2):
    n_blk = N_NODES // _BN
    row_spec = pl.BlockSpec((_BN, D_NODE), lambda i: (i, 0))
    const = lambda shape: pl.BlockSpec(shape, lambda i: (0, 0))
    return pl.pallas_call(
        _node_body,
        grid=(n_blk,),
        in_specs=[
            row_spec,
            pl.BlockSpec((_NC, _BN, D_NODE), lambda i: (0, i, 0)),
            pl.BlockSpec((_NC, _BN, D_NODE), lambda i: (0, i, 0)),
            const((D_NODE, D_NODE)),
            const((D_NODE, D_NODE)),
            const((1, D_NODE)),
            const((1, D_NODE)),
            const((1, D_NODE)),
            const((D_NODE, D_NODE)),
            const((1, D_NODE)),
            const((1, D_NODE)),
            const((1, D_NODE)),
        ],
        out_specs=row_spec,
        out_shape=jax.ShapeDtypeStruct((N_NODES, D_NODE), jnp.float32),
    )(nf, agg_a, agg_b, nwx, nwa, nb1, ng1, nbt1, nw2, nb2, ng2, nbt2)


def kernel(node_features, edge_features,
           eW1, eb1, eg1, ebt1, eW2, eb2, eg2, ebt2,
           nW1, nb1, ng1, nbt1, nW2, nb2, ng2, nbt2,
           edge_index):
    ei = edge_index.astype(jnp.int32)
    row = ei[0]
    col = ei[1]

    wsd = eW1[0:D_NODE] + eW1[2 * D_NODE:3 * D_NODE]
    wtd = eW1[D_NODE:2 * D_NODE] - eW1[2 * D_NODE:3 * D_NODE]
    wdc = eW1[3 * D_NODE:3 * D_NODE + 2]
    we = eW1[3 * D_NODE + 2:]

    half = N_EDGES // 2
    e_halves = []
    aggs = []
    for h in range(2):
        row_h = lax.slice(row, (h * half,), ((h + 1) * half,))
        col_h = lax.slice(col, (h * half,), ((h + 1) * half,))
        ef_h = lax.slice(edge_features, (h * half, 0), ((h + 1) * half, D_EDGE))
        src_h, tgt_h = _sc_gather(node_features, row_h, col_h)
        e_h = _edge_mlp(
            src_h, tgt_h, ef_h, wsd, wtd, we, wdc,
            eb1.reshape(1, -1), eg1.reshape(1, -1), ebt1.reshape(1, -1),
            eW2, eb2.reshape(1, -1), eg2.reshape(1, -1), ebt2.reshape(1, -1))
        e_halves.append(e_h)
        aggs.append(_sc_scatter(e_h, col_h))
    e_out = jnp.concatenate(e_halves, axis=0)
    x_out = _node_mlp(
        node_features, aggs[0], aggs[1], nW1[0:D_NODE], nW1[D_NODE:],
        nb1.reshape(1, -1), ng1.reshape(1, -1), nbt1.reshape(1, -1),
        nW2, nb2.reshape(1, -1), ng2.reshape(1, -1), nbt2.reshape(1, -1))
    return (x_out, e_out)


# speedup vs baseline: 1.7252x; 1.0446x over previous
import functools

import jax
import jax.numpy as jnp
import numpy as np
from jax import lax
from jax.experimental import pallas as pl
from jax.experimental.pallas import tpu as pltpu
from jax.experimental.pallas import tpu_sc as plsc

N_NODES = 10000
N_EDGES = 320000
D_NODE = 128
D_EDGE = 16

_NC, _NS = 2, 16
_NW = _NC * _NS
_CH = 128
_NCHUNK = N_EDGES // _CH
_SLOTS = -(-_NCHUNK // _NW)
_ZCH = 40
_NZ = N_NODES // _ZCH
_ZSLOTS = -(-_NZ // _NS)

_MESH = dict(core_axis_name="c", subcore_axis_name="s")


def _sc_gather(node_features, row, col):
    n_e = row.shape[0]
    n_chunk = n_e // _CH
    slots = -(-n_chunk // _NW)

    @functools.partial(
        pl.kernel,
        out_type=(
            jax.ShapeDtypeStruct((n_e, D_NODE), jnp.float32),
            jax.ShapeDtypeStruct((n_e, D_NODE), jnp.float32),
        ),
        scratch_types=[
            pltpu.VMEM((2, _CH), jnp.int32),
            pltpu.VMEM((2, _CH), jnp.int32),
            pltpu.VMEM((2, _CH, D_NODE), jnp.float32),
            pltpu.VMEM((2, _CH, D_NODE), jnp.float32),
            pltpu.SemaphoreType.DMA,
            pltpu.SemaphoreType.DMA,
        ],
        mesh=plsc.VectorSubcoreMesh(**_MESH),
    )
    def gather_k(nf, row_h, col_h, src_o, tgt_o, idx_r, idx_c, buf_s, buf_t,
                 sem0, sem1):
        wid = lax.axis_index("s") * _NC + lax.axis_index("c")
        sems = (sem0, sem1)

        def body(gg, c):
            for r in range(2):
                g = 2 * gg + r
                c0 = wid + _NW * g
                c1 = c0 - _NW
                c2 = c0 - 2 * _NW

                @pl.when((g >= 2) & (c2 < n_chunk))
                def _():
                    base = c2 * _CH
                    pltpu.make_async_copy(
                        src_o.at[pl.ds(base, _CH)], buf_s.at[r], sems[r]).wait()
                    pltpu.make_async_copy(
                        tgt_o.at[pl.ds(base, _CH)], buf_t.at[r], sems[r]).wait()

                @pl.when(c0 < n_chunk)
                def _():
                    base = c0 * _CH
                    pltpu.sync_copy(row_h.at[pl.ds(base, _CH)], idx_r.at[r])
                    pltpu.sync_copy(col_h.at[pl.ds(base, _CH)], idx_c.at[r])
                    pltpu.async_copy(nf.at[idx_r.at[r]], buf_s.at[r], sems[r])
                    pltpu.async_copy(nf.at[idx_c.at[r]], buf_t.at[r], sems[r])

                @pl.when((g >= 1) & (c1 < n_chunk))
                def _():
                    q = 1 - r
                    base = c1 * _CH
                    pltpu.make_async_copy(
                        nf.at[idx_r.at[q]], buf_s.at[q], sems[q]).wait()
                    pltpu.make_async_copy(
                        nf.at[idx_c.at[q]], buf_t.at[q], sems[q]).wait()
                    pltpu.async_copy(
                        buf_s.at[q], src_o.at[pl.ds(base, _CH)], sems[q])
                    pltpu.async_copy(
                        buf_t.at[q], tgt_o.at[pl.ds(base, _CH)], sems[q])

            return c

        lax.fori_loop(0, (slots + 2 + 1) // 2, body, 0)

    return gather_k(node_features, row, col)


def _sc_scatter(e_out, col):
    n_e = col.shape[0]
    n_chunk = n_e // _CH
    slots = -(-n_chunk // _NW)

    @functools.partial(
        pl.kernel,
        out_type=jax.ShapeDtypeStruct((_NC, N_NODES, D_NODE), jnp.float32),
        scratch_types=[
            pltpu.VMEM((2, _CH), jnp.int32),
            pltpu.VMEM((2, _CH, D_NODE), jnp.float32),
            pltpu.VMEM((_ZCH, D_NODE), jnp.float32),
            pltpu.VMEM_SHARED((N_NODES, D_NODE), jnp.float32),
            pltpu.SemaphoreType.DMA,
            pltpu.SemaphoreType.DMA,
        ],
        mesh=plsc.VectorSubcoreMesh(**_MESH),
    )
    def scatter_k(eout_h, col_h, agg_o, idx_c, buf, zbuf, agg_sh, sem0, sem1):
        cid = lax.axis_index("c")
        sid = lax.axis_index("s")
        wid = sid * _NC + cid

        def zrow(i, c):
            def zcol(j, c2):
                zbuf[i, pl.ds(j * 16, 16)] = jnp.zeros((16,), jnp.float32)
                return c2

            return lax.fori_loop(0, D_NODE // 16, zcol, c)

        lax.fori_loop(0, _ZCH, zrow, 0)

        def zcp(k, c):
            zc = sid + _NS * k

            @pl.when(zc < _NZ)
            def _():
                pltpu.sync_copy(zbuf, agg_sh.at[pl.ds(zc * _ZCH, _ZCH)])

            return c

        lax.fori_loop(0, _ZSLOTS, zcp, 0)
        plsc.subcore_barrier()

        sems = (sem0, sem1)

        def body(gg, c):
            for r in range(2):
                g = 2 * gg + r
                c0 = wid + _NW * g
                c1 = c0 - _NW

                @pl.when(c0 < n_chunk)
                def _():
                    base = c0 * _CH
                    pltpu.async_copy(
                        col_h.at[pl.ds(base, _CH)], idx_c.at[r], sems[r])
                    pltpu.async_copy(
                        eout_h.at[pl.ds(base, _CH)], buf.at[r], sems[r])

                @pl.when((g >= 1) & (c1 < n_chunk))
                def _():
                    q = 1 - r
                    base = c1 * _CH
                    pltpu.make_async_copy(
                        col_h.at[pl.ds(base, _CH)], idx_c.at[q], sems[q]).wait()
                    pltpu.make_async_copy(
                        eout_h.at[pl.ds(base, _CH)], buf.at[q], sems[q]).wait()
                    pltpu.sync_copy(buf.at[q], agg_sh.at[idx_c.at[q]], add=True)

            return c

        lax.fori_loop(0, (slots + 1 + 1) // 2, body, 0)
        plsc.subcore_barrier()

        def cp(k, c):
            zc = sid + _NS * k

            @pl.when(zc < _NZ)
            def _():
                off = zc * _ZCH
                pltpu.sync_copy(agg_sh.at[pl.ds(off, _ZCH)], zbuf)
                pltpu.sync_copy(zbuf, agg_o.at[cid, pl.ds(off, _ZCH)])

            return c

        lax.fori_loop(0, _ZSLOTS, cp, 0)

    return scatter_k(e_out, col)


def _rowsum(x):
    return jnp.sum(x, axis=1, keepdims=True)


def _ln_gelu(x, g, b):
    m = _rowsum(x) * (1.0 / x.shape[1])
    c = x - m
    v = _rowsum(c * c) * (1.0 / x.shape[1])
    y = c * lax.rsqrt(v + 1e-5) * g + b
    return 0.5 * y * (1.0 + lax.erf(y * (1.0 / np.sqrt(2.0))))


_BE = 2000


def _edge_body(src, tgt, ef, wsd, wtd, we, wdc, eb1, eg1, ebt1,
               ew2, eb2, eg2, ebt2, out):
    s = src[...]
    t = tgt[...]
    st = _rowsum(s * t)
    ss = _rowsum(s * s)
    tt = _rowsum(t * t)
    d2 = jnp.maximum(ss + tt - 2.0 * st, 0.0)
    dist = jnp.sqrt(d2 + 1e-12)
    cos = st / (jnp.sqrt(ss + 1e-12) * jnp.sqrt(tt + 1e-12))
    h = jnp.dot(s, wsd[...], preferred_element_type=jnp.float32)
    h = h + jnp.dot(t, wtd[...], preferred_element_type=jnp.float32)
    h = h + jnp.dot(ef[...], we[...], preferred_element_type=jnp.float32)
    h = h + dist * wdc[0:1, :] + cos * wdc[1:2, :] + eb1[...]
    h = _ln_gelu(h, eg1[...], ebt1[...])
    h2 = jnp.dot(h, ew2[...], preferred_element_type=jnp.float32) + eb2[...]
    out[...] = _ln_gelu(h2, eg2[...], ebt2[...])


def _edge_mlp(src, tgt, ef, wsd, wtd, we, wdc, eb1, eg1, ebt1,
              ew2, eb2, eg2, ebt2):
    n_blk = src.shape[0] // _BE
    row_spec = pl.BlockSpec((_BE, D_NODE), lambda i: (i, 0))
    const = lambda shape: pl.BlockSpec(shape, lambda i: (0, 0))
    return pl.pallas_call(
        _edge_body,
        grid=(n_blk,),
        in_specs=[
            row_spec,
            row_spec,
            pl.BlockSpec((_BE, D_EDGE), lambda i: (i, 0)),
            const((D_NODE, D_NODE)),
            const((D_NODE, D_NODE)),
            const((D_EDGE, D_NODE)),
            const((2, D_NODE)),
            const((1, D_NODE)),
            const((1, D_NODE)),
            const((1, D_NODE)),
            const((D_NODE, D_NODE)),
            const((1, D_NODE)),
            const((1, D_NODE)),
            const((1, D_NODE)),
        ],
        out_specs=row_spec,
        out_shape=jax.ShapeDtypeStruct((src.shape[0], D_NODE), jnp.float32),
    )(src, tgt, ef, wsd, wtd, we, wdc, eb1, eg1, ebt1, ew2, eb2, eg2, ebt2)


_BN = 1000


def _node_body(nf, agg_a, agg_b, nwx, nwa, nb1, ng1, nbt1, nw2, nb2, ng2, nbt2, out):
    x = nf[...]
    a = agg_a[0] + agg_a[1] + agg_b[0] + agg_b[1]
    h = jnp.dot(x, nwx[...], preferred_element_type=jnp.float32)
    h = h + jnp.dot(a, nwa[...], preferred_element_type=jnp.float32) + nb1[...]
    h = _ln_gelu(h, ng1[...], nbt1[...])
    h2 = jnp.dot(h, nw2[...], preferred_element_type=jnp.float32) + nb2[...]
    out[...] = _ln_gelu(h2, ng2[...], nbt2[...])


def _node_mlp(nf, agg_a, agg_b, nwx, nwa, nb1, ng1, nbt1, nw2, nb2, ng2, nbt2):
    n_blk = N_NODES // _BN
    row_spec = pl.BlockSpec((_BN, D_NODE), lambda i: (i, 0))
    const = lambda shape: pl.BlockSpec(shape, lambda i: (0, 0))
    return pl.pallas_call(
        _node_body,
        grid=(n_blk,),
        in_specs=[
            row_spec,
            pl.BlockSpec((_NC, _BN, D_NODE), lambda i: (0, i, 0)),
            pl.BlockSpec((_NC, _BN, D_NODE), lambda i: (0, i, 0)),
            const((D_NODE, D_NODE)),
            const((D_NODE, D_NODE)),
            const((1, D_NODE)),
            const((1, D_NODE)),
            const((1, D_NODE)),
            const((D_NODE, D_NODE)),
            const((1, D_NODE)),
            const((1, D_NODE)),
            const((1, D_NODE)),
        ],
        out_specs=row_spec,
        out_shape=jax.ShapeDtypeStruct((N_NODES, D_NODE), jnp.float32),
    )(nf, agg_a, agg_b, nwx, nwa, nb1, ng1, nbt1, nw2, nb2, ng2, nbt2)


def kernel(node_features, edge_features,
           eW1, eb1, eg1, ebt1, eW2, eb2, eg2, ebt2,
           nW1, nb1, ng1, nbt1, nW2, nb2, ng2, nbt2,
           edge_index):
    ei = edge_index.astype(jnp.int32)
    row = ei[0]
    col = ei[1]

    wsd = eW1[0:D_NODE] + eW1[2 * D_NODE:3 * D_NODE]
    wtd = eW1[D_NODE:2 * D_NODE] - eW1[2 * D_NODE:3 * D_NODE]
    wdc = eW1[3 * D_NODE:3 * D_NODE + 2]
    we = eW1[3 * D_NODE + 2:]

    half = N_EDGES // 2
    e_halves = []
    aggs = []
    for h in range(2):
        row_h = lax.slice(row, (h * half,), ((h + 1) * half,))
        col_h = lax.slice(col, (h * half,), ((h + 1) * half,))
        ef_h = lax.slice(edge_features, (h * half, 0), ((h + 1) * half, D_EDGE))
        src_h, tgt_h = _sc_gather(node_features, row_h, col_h)
        e_h = _edge_mlp(
            src_h, tgt_h, ef_h, wsd, wtd, we, wdc,
            eb1.reshape(1, -1), eg1.reshape(1, -1), ebt1.reshape(1, -1),
            eW2, eb2.reshape(1, -1), eg2.reshape(1, -1), ebt2.reshape(1, -1))
        e_halves.append(e_h)
        aggs.append(_sc_scatter(e_h, col_h))
    e_out = jnp.concatenate(e_halves, axis=0)
    x_out = _node_mlp(
        node_features, aggs[0], aggs[1], nW1[0:D_NODE], nW1[D_NODE:],
        nb1.reshape(1, -1), ng1.reshape(1, -1), nbt1.reshape(1, -1),
        nW2, nb2.reshape(1, -1), ng2.reshape(1, -1), nbt2.reshape(1, -1))
    return (x_out, e_out)


# speedup vs baseline: 1.8035x; 1.0454x over previous
import functools

import jax
import jax.numpy as jnp
import numpy as np
from jax import lax
from jax.experimental import pallas as pl
from jax.experimental.pallas import tpu as pltpu
from jax.experimental.pallas import tpu_sc as plsc

N_NODES = 10000
N_EDGES = 320000
D_NODE = 128
D_EDGE = 16

_NC, _NS = 2, 16
_NW = _NC * _NS
_CH = 128
_NCHUNK = N_EDGES // _CH
_SLOTS = -(-_NCHUNK // _NW)
_ZCH = 40
_NZ = N_NODES // _ZCH
_ZSLOTS = -(-_NZ // _NS)

_MESH = dict(core_axis_name="c", subcore_axis_name="s")


def _sc_gather(node_features, row, col, off, n_e):
    n_chunk = n_e // _CH
    slots = -(-n_chunk // _NW)

    @functools.partial(
        pl.kernel,
        out_type=(
            jax.ShapeDtypeStruct((n_e, D_NODE), jnp.float32),
            jax.ShapeDtypeStruct((n_e, D_NODE), jnp.float32),
        ),
        scratch_types=[
            pltpu.VMEM((2, _CH), jnp.int32),
            pltpu.VMEM((2, _CH), jnp.int32),
            pltpu.VMEM((2, _CH, D_NODE), jnp.float32),
            pltpu.VMEM((2, _CH, D_NODE), jnp.float32),
            pltpu.SemaphoreType.DMA,
            pltpu.SemaphoreType.DMA,
        ],
        mesh=plsc.VectorSubcoreMesh(**_MESH),
    )
    def gather_k(nf, row_h, col_h, src_o, tgt_o, idx_r, idx_c, buf_s, buf_t,
                 sem0, sem1):
        wid = lax.axis_index("s") * _NC + lax.axis_index("c")
        sems = (sem0, sem1)

        def body(gg, c):
            for r in range(2):
                g = 2 * gg + r
                c0 = wid + _NW * g
                c1 = c0 - _NW
                c2 = c0 - 2 * _NW

                @pl.when((g >= 2) & (c2 < n_chunk))
                def _():
                    base = c2 * _CH
                    pltpu.make_async_copy(
                        src_o.at[pl.ds(base, _CH)], buf_s.at[r], sems[r]).wait()
                    pltpu.make_async_copy(
                        tgt_o.at[pl.ds(base, _CH)], buf_t.at[r], sems[r]).wait()

                @pl.when(c0 < n_chunk)
                def _():
                    gbase = (off + c0) * _CH
                    pltpu.sync_copy(row_h.at[pl.ds(gbase, _CH)], idx_r.at[r])
                    pltpu.sync_copy(col_h.at[pl.ds(gbase, _CH)], idx_c.at[r])
                    pltpu.async_copy(nf.at[idx_r.at[r]], buf_s.at[r], sems[r])
                    pltpu.async_copy(nf.at[idx_c.at[r]], buf_t.at[r], sems[r])

                @pl.when((g >= 1) & (c1 < n_chunk))
                def _():
                    q = 1 - r
                    base = c1 * _CH
                    pltpu.make_async_copy(
                        nf.at[idx_r.at[q]], buf_s.at[q], sems[q]).wait()
                    pltpu.make_async_copy(
                        nf.at[idx_c.at[q]], buf_t.at[q], sems[q]).wait()
                    pltpu.async_copy(
                        buf_s.at[q], src_o.at[pl.ds(base, _CH)], sems[q])
                    pltpu.async_copy(
                        buf_t.at[q], tgt_o.at[pl.ds(base, _CH)], sems[q])

            return c

        lax.fori_loop(0, (slots + 2 + 1) // 2, body, 0)

    return gather_k(node_features, row, col)


def _sc_scatter(e_out, col, off):
    n_e = e_out.shape[0]
    n_chunk = n_e // _CH
    slots = -(-n_chunk // _NW)

    @functools.partial(
        pl.kernel,
        out_type=jax.ShapeDtypeStruct((_NC, N_NODES, D_NODE), jnp.float32),
        scratch_types=[
            pltpu.VMEM((2, _CH), jnp.int32),
            pltpu.VMEM((2, _CH, D_NODE), jnp.float32),
            pltpu.VMEM((_ZCH, D_NODE), jnp.float32),
            pltpu.VMEM_SHARED((N_NODES, D_NODE), jnp.float32),
            pltpu.SemaphoreType.DMA,
            pltpu.SemaphoreType.DMA,
        ],
        mesh=plsc.VectorSubcoreMesh(**_MESH),
    )
    def scatter_k(eout_h, col_h, agg_o, idx_c, buf, zbuf, agg_sh, sem0, sem1):
        cid = lax.axis_index("c")
        sid = lax.axis_index("s")
        wid = sid * _NC + cid

        def zrow(i, c):
            def zcol(j, c2):
                zbuf[i, pl.ds(j * 16, 16)] = jnp.zeros((16,), jnp.float32)
                return c2

            return lax.fori_loop(0, D_NODE // 16, zcol, c)

        lax.fori_loop(0, _ZCH, zrow, 0)

        def zcp(k, c):
            zc = sid + _NS * k

            @pl.when(zc < _NZ)
            def _():
                pltpu.sync_copy(zbuf, agg_sh.at[pl.ds(zc * _ZCH, _ZCH)])

            return c

        lax.fori_loop(0, _ZSLOTS, zcp, 0)
        plsc.subcore_barrier()

        sems = (sem0, sem1)

        def body(gg, c):
            for r in range(2):
                g = 2 * gg + r
                c0 = wid + _NW * g
                c1 = c0 - _NW

                @pl.when(c0 < n_chunk)
                def _():
                    base = c0 * _CH
                    pltpu.async_copy(
                        col_h.at[pl.ds((off + c0) * _CH, _CH)], idx_c.at[r],
                        sems[r])
                    pltpu.async_copy(
                        eout_h.at[pl.ds(base, _CH)], buf.at[r], sems[r])

                @pl.when((g >= 1) & (c1 < n_chunk))
                def _():
                    q = 1 - r
                    base = c1 * _CH
                    pltpu.make_async_copy(
                        col_h.at[pl.ds((off + c1) * _CH, _CH)], idx_c.at[q],
                        sems[q]).wait()
                    pltpu.make_async_copy(
                        eout_h.at[pl.ds(base, _CH)], buf.at[q], sems[q]).wait()
                    pltpu.sync_copy(buf.at[q], agg_sh.at[idx_c.at[q]], add=True)

            return c

        lax.fori_loop(0, (slots + 1 + 1) // 2, body, 0)
        plsc.subcore_barrier()

        def cp(k, c):
            zc = sid + _NS * k

            @pl.when(zc < _NZ)
            def _():
                off = zc * _ZCH
                pltpu.sync_copy(agg_sh.at[pl.ds(off, _ZCH)], zbuf)
                pltpu.sync_copy(zbuf, agg_o.at[cid, pl.ds(off, _ZCH)])

            return c

        lax.fori_loop(0, _ZSLOTS, cp, 0)

    return scatter_k(e_out, col)


def _rowsum(x):
    return jnp.sum(x, axis=1, keepdims=True)


def _ln_gelu(x, g, b):
    m = _rowsum(x) * (1.0 / x.shape[1])
    c = x - m
    v = _rowsum(c * c) * (1.0 / x.shape[1])
    y = c * lax.rsqrt(v + 1e-5) * g + b
    return 0.5 * y * (1.0 + lax.erf(y * (1.0 / np.sqrt(2.0))))


_BE = 2000


def _edge_body(src, tgt, ef, wsd, wtd, we, wdc, eb1, eg1, ebt1,
               ew2, eb2, eg2, ebt2, out):
    s = src[...]
    t = tgt[...]
    st = _rowsum(s * t)
    ss = _rowsum(s * s)
    tt = _rowsum(t * t)
    d2 = jnp.maximum(ss + tt - 2.0 * st, 0.0)
    dist = jnp.sqrt(d2 + 1e-12)
    cos = st / (jnp.sqrt(ss + 1e-12) * jnp.sqrt(tt + 1e-12))
    h = jnp.dot(s, wsd[...], preferred_element_type=jnp.float32)
    h = h + jnp.dot(t, wtd[...], preferred_element_type=jnp.float32)
    h = h + jnp.dot(ef[...], we[...], preferred_element_type=jnp.float32)
    h = h + dist * wdc[0:1, :] + cos * wdc[1:2, :] + eb1[...]
    h = _ln_gelu(h, eg1[...], ebt1[...])
    h2 = jnp.dot(h, ew2[...], preferred_element_type=jnp.float32) + eb2[...]
    out[...] = _ln_gelu(h2, eg2[...], ebt2[...])


def _edge_mlp(src, tgt, ef, wsd, wtd, we, wdc, eb1, eg1, ebt1,
              ew2, eb2, eg2, ebt2, blk_off):
    n_blk = src.shape[0] // _BE
    row_spec = pl.BlockSpec((_BE, D_NODE), lambda i: (i, 0))
    const = lambda shape: pl.BlockSpec(shape, lambda i: (0, 0))
    return pl.pallas_call(
        _edge_body,
        grid=(n_blk,),
        in_specs=[
            row_spec,
            row_spec,
            pl.BlockSpec((_BE, D_EDGE), lambda i: (blk_off + i, 0)),
            const((D_NODE, D_NODE)),
            const((D_NODE, D_NODE)),
            const((D_EDGE, D_NODE)),
            const((2, D_NODE)),
            const((1, D_NODE)),
            const((1, D_NODE)),
            const((1, D_NODE)),
            const((D_NODE, D_NODE)),
            const((1, D_NODE)),
            const((1, D_NODE)),
            const((1, D_NODE)),
        ],
        out_specs=row_spec,
        out_shape=jax.ShapeDtypeStruct((src.shape[0], D_NODE), jnp.float32),
    )(src, tgt, ef, wsd, wtd, we, wdc, eb1, eg1, ebt1, ew2, eb2, eg2, ebt2)


_BN = 1000


def _node_mlp(nf, agg_list, nwx, nwa, nb1, ng1, nbt1, nw2, nb2, ng2, nbt2):
    n_parts = len(agg_list)
    n_blk = N_NODES // _BN
    row_spec = pl.BlockSpec((_BN, D_NODE), lambda i: (i, 0))
    const = lambda shape: pl.BlockSpec(shape, lambda i: (0, 0))

    def body(*refs):
        nf_r = refs[0]
        aggs = refs[1:1 + n_parts]
        (nwx_r, nwa_r, nb1_r, ng1_r, nbt1_r, nw2_r, nb2_r, ng2_r, nbt2_r,
         out) = refs[1 + n_parts:]
        x = nf_r[...]
        a = aggs[0][0] + aggs[0][1]
        for ar in aggs[1:]:
            a = a + ar[0] + ar[1]
        h = jnp.dot(x, nwx_r[...], preferred_element_type=jnp.float32)
        h = h + jnp.dot(a, nwa_r[...], preferred_element_type=jnp.float32)
        h = h + nb1_r[...]
        h = _ln_gelu(h, ng1_r[...], nbt1_r[...])
        h2 = jnp.dot(h, nw2_r[...], preferred_element_type=jnp.float32)
        h2 = h2 + nb2_r[...]
        out[...] = _ln_gelu(h2, ng2_r[...], nbt2_r[...])

    return pl.pallas_call(
        body,
        grid=(n_blk,),
        in_specs=[row_spec]
        + [pl.BlockSpec((_NC, _BN, D_NODE), lambda i: (0, i, 0))] * n_parts
        + [
            const((D_NODE, D_NODE)),
            const((D_NODE, D_NODE)),
            const((1, D_NODE)),
            const((1, D_NODE)),
            const((1, D_NODE)),
            const((D_NODE, D_NODE)),
            const((1, D_NODE)),
            const((1, D_NODE)),
            const((1, D_NODE)),
        ],
        out_specs=row_spec,
        out_shape=jax.ShapeDtypeStruct((N_NODES, D_NODE), jnp.float32),
    )(nf, *agg_list, nwx, nwa, nb1, ng1, nbt1, nw2, nb2, ng2, nbt2)


def kernel(node_features, edge_features,
           eW1, eb1, eg1, ebt1, eW2, eb2, eg2, ebt2,
           nW1, nb1, ng1, nbt1, nW2, nb2, ng2, nbt2,
           edge_index):
    ei = edge_index.astype(jnp.int32)
    row = ei[0]
    col = ei[1]

    wsd = eW1[0:D_NODE] + eW1[2 * D_NODE:3 * D_NODE]
    wtd = eW1[D_NODE:2 * D_NODE] - eW1[2 * D_NODE:3 * D_NODE]
    wdc = eW1[3 * D_NODE:3 * D_NODE + 2]
    we = eW1[3 * D_NODE + 2:]

    n_split = 2
    part = N_EDGES // n_split
    e_parts = []
    aggs = []
    for h in range(n_split):
        off = h * part // _CH
        src_h, tgt_h = _sc_gather(node_features, row, col, off, part)
        e_h = _edge_mlp(
            src_h, tgt_h, edge_features, wsd, wtd, we, wdc,
            eb1.reshape(1, -1), eg1.reshape(1, -1), ebt1.reshape(1, -1),
            eW2, eb2.reshape(1, -1), eg2.reshape(1, -1), ebt2.reshape(1, -1),
            h * part // _BE)
        e_parts.append(e_h)
        aggs.append(_sc_scatter(e_h, col, off))
    e_out = jnp.concatenate(e_parts, axis=0)
    x_out = _node_mlp(
        node_features, aggs, nW1[0:D_NODE], nW1[D_NODE:],
        nb1.reshape(1, -1), ng1.reshape(1, -1), nbt1.reshape(1, -1),
        nW2, nb2.reshape(1, -1), ng2.reshape(1, -1), nbt2.reshape(1, -1))
    return (x_out, e_out)
